# baseline (speedup 1.0000x reference)
import jax
import jax.numpy as jnp
from jax.experimental import pallas as pl


def _fe_block(x_ref, w_ref, o_ref):
    bb, t, f2 = x_ref.shape
    d = w_ref.shape[1]
    x = x_ref[...].reshape(bb * t, f2)
    y = jnp.dot(x, w_ref[...], preferred_element_type=jnp.float32)
    o_ref[...] = y.reshape(bb, t, d)


def kernel(inputs, lookup_table_real, lookup_table_imag):
    B, T, F2 = inputs.shape
    half = lookup_table_real.shape[1]
    D = 2 * half
    F = F2 // 2

    w = jnp.zeros((F2, D), jnp.float32)
    w = w.at[:F, :half].set(lookup_table_real)
    w = w.at[F:, half:].set(lookup_table_imag)

    BB = 16
    assert B % BB == 0
    grid = (B // BB,)

    out = pl.pallas_call(
        _fe_block,
        grid=grid,
        in_specs=[
            pl.BlockSpec((BB, T, F2), lambda i: (i, 0, 0)),
            pl.BlockSpec((F2, D), lambda i: (0, 0)),
        ],
        out_specs=pl.BlockSpec((BB, T, D), lambda i: (i, 0, 0)),
        out_shape=jax.ShapeDtypeStruct((B, T, D), jnp.float32),
    )(inputs, w)
    return out


# speedup vs baseline: 1.1610x; 1.1610x over previous
import jax
import jax.numpy as jnp
from jax.experimental import pallas as pl


def _fe_block(x_ref, w_ref, o_ref):
    o_ref[...] = jnp.dot(x_ref[...], w_ref[...],
                         preferred_element_type=jnp.float32)


def kernel(inputs, lookup_table_real, lookup_table_imag):
    B, T, F2 = inputs.shape
    half = lookup_table_real.shape[1]
    D = 2 * half
    F = F2 // 2
    rows = B * T
    x = inputs.reshape(rows, F2)

    w = jnp.zeros((F2, D), jnp.float32)
    w = w.at[:F, :half].set(lookup_table_real)
    w = w.at[F:, half:].set(lookup_table_imag)

    R = 8192
    assert rows % R == 0
    grid = (rows // R,)

    out = pl.pallas_call(
        _fe_block,
        grid=grid,
        in_specs=[
            pl.BlockSpec((R, F2), lambda i: (i, 0)),
            pl.BlockSpec((F2, D), lambda i: (0, 0)),
        ],
        out_specs=pl.BlockSpec((R, D), lambda i: (i, 0)),
        out_shape=jax.ShapeDtypeStruct((rows, D), jnp.float32),
    )(x, w)
    return out.reshape(B, T, D)


# speedup vs baseline: 1.1660x; 1.0043x over previous
import jax
import jax.numpy as jnp
from jax.experimental import pallas as pl


def _fe_block(x_ref, w_ref, o_ref):
    o_ref[...] = jnp.dot(x_ref[...], w_ref[...],
                         preferred_element_type=jnp.float32)


def kernel(inputs, lookup_table_real, lookup_table_imag):
    B, T, F2 = inputs.shape
    half = lookup_table_real.shape[1]
    D = 2 * half
    F = F2 // 2
    rows = B * T
    x = inputs.reshape(rows, F2)

    w = jnp.zeros((F2, D), jnp.float32)
    w = w.at[:F, :half].set(lookup_table_real)
    w = w.at[F:, half:].set(lookup_table_imag)

    R = 12800
    assert rows % R == 0
    grid = (rows // R,)

    out = pl.pallas_call(
        _fe_block,
        grid=grid,
        in_specs=[
            pl.BlockSpec((R, F2), lambda i: (i, 0)),
            pl.BlockSpec((F2, D), lambda i: (0, 0)),
        ],
        out_specs=pl.BlockSpec((R, D), lambda i: (i, 0)),
        out_shape=jax.ShapeDtypeStruct((rows, D), jnp.float32),
    )(x, w)
    return out.reshape(B, T, D)
